# baseline (speedup 1.0000x reference)
import functools

import jax
import jax.numpy as jnp
from jax import lax
from jax.experimental import pallas as pl
from jax.experimental.pallas import tpu as pltpu
from jax.experimental.pallas import tpu_sc as plsc

VOCAB = 100000
HIDDEN = 768
MAX_POS = 8192
BATCH = 4
SEQ = 8192

NC = 2
NS = 16
NW = NC * NS
L = 16

S_PER_W = SEQ // NW
CH = 16
HF = CH // 2
NCH = S_PER_W // CH
NH = NCH // 2
UNITS = HIDDEN // L
NB = 8
AH = 4

_mesh = plsc.VectorSubcoreMesh(
    core_axis_name="c", subcore_axis_name="s", num_cores=NC, num_subcores=NS
)


@functools.partial(
    pl.kernel,
    out_type=jax.ShapeDtypeStruct((BATCH, SEQ, HIDDEN), jnp.float32),
    mesh=_mesh,
    scratch_types=[
        pltpu.VMEM((BATCH, S_PER_W), jnp.int32),
        pltpu.VMEM((CH, HIDDEN), jnp.float32),
        pltpu.VMEM((CH, HIDDEN), jnp.float32),
    ] + [pltpu.VMEM((CH, HIDDEN), jnp.float32)] * 8
      + [pltpu.SemaphoreType.DMA] * 18,
)
def _embed(idx_hbm, tok_hbm, pos_hbm, out_hbm, idx_v, *bufs_and_sems):
    pbuf = bufs_and_sems[0:2]
    rbuf = bufs_and_sems[2:10]
    psem = bufs_and_sems[10:12]
    gsem = bufs_and_sems[12:20]
    ssem = bufs_and_sems[20:28]
    wid = lax.axis_index("s") * NC + lax.axis_index("c")
    s_base = wid * S_PER_W

    def gather(c, b, buf):
        return pltpu.async_copy(
            tok_hbm.at[idx_v.at[b, pl.ds(c * CH, CH)]], rbuf[buf], gsem[buf]
        )

    def gather_wait(buf):
        pltpu.make_async_copy(
            tok_hbm.at[idx_v.at[0, pl.ds(0, CH)]], rbuf[buf], gsem[buf]
        ).wait()

    def store_wait(buf):
        pltpu.make_async_copy(
            rbuf[buf], out_hbm.at[0, pl.ds(s_base, CH)], ssem[buf]
        ).wait()

    def pos_load(c, buf):
        return pltpu.async_copy(
            pos_hbm.at[pl.ds(s_base + c * CH, CH)], pbuf[buf], psem[buf]
        )

    pltpu.sync_copy(idx_hbm.at[0, pl.ds(s_base, S_PER_W)], idx_v.at[0])
    pos_load(0, 0)
    pos_load(1, 1)
    gather(0, 0, 0)
    pltpu.sync_copy(idx_hbm.at[1, pl.ds(s_base, S_PER_W)], idx_v.at[1])
    gather(0, 1, 1)
    pltpu.sync_copy(idx_hbm.at[2, pl.ds(s_base, S_PER_W)], idx_v.at[2])
    gather(0, 2, 2)
    pltpu.sync_copy(idx_hbm.at[3, pl.ds(s_base, S_PER_W)], idx_v.at[3])
    gather(0, 3, 3)

    def body(h, _):
        store_desc = [None] * NB
        gather_desc = [None] * NB
        for k in range(8):
            rb = k % NB
            fb = (k + AH) % NB
            pb = k // 4
            b = k % 4
            c = 2 * h + k // 4
            s0 = s_base + c * CH

            if k < AH:
                @pl.when(h > 0)
                def _():
                    store_wait(fb)
            else:
                store_desc[fb].wait()

            if k < 8 - AH:
                gather_desc[fb] = gather(2 * h + (k + AH) // 4, (k + AH) % 4,
                                         fb)
            else:
                @pl.when(h < NH - 1)
                def _():
                    gather(2 * h + 2, k - (8 - AH), fb)

            if k < AH:
                gather_wait(rb)
            else:
                gather_desc[rb].wait()

            if k == 0 or k == 4:
                pltpu.make_async_copy(
                    pos_hbm.at[pl.ds(s_base, CH)], pbuf[pb], psem[pb]
                ).wait()

            def add_rows(lo, hi, _rb=rb, _pb=pb):
                @plsc.parallel_loop(lo, hi)
                def _(r):
                    for j in range(UNITS):
                        plsc.addupdate(
                            rbuf[_rb].at[r, pl.ds(j * L, L)],
                            pbuf[_pb][r, pl.ds(j * L, L)],
                        )

            for half in range(2):
                add_rows(half * HF, (half + 1) * HF)
                pltpu.async_copy(
                    rbuf[rb].at[pl.ds(half * HF, HF)],
                    out_hbm.at[b, pl.ds(s0 + half * HF, HF)],
                    ssem[rb],
                )
            store_desc[rb] = pltpu.make_async_copy(
                rbuf[rb], out_hbm.at[b, pl.ds(s0, CH)], ssem[rb]
            )

            if k == 3 or k == 7:
                @pl.when(h < NH - 1)
                def _():
                    pos_load(2 * h + 2 + k // 4, pb)
        return 0

    lax.fori_loop(0, NH, body, 0)

    for buf in range(AH, 8):
        store_wait(buf)


def kernel(input_ids, tok_embed, pos_embed):
    return _embed(input_ids.astype(jnp.int32), tok_embed, pos_embed)


# speedup vs baseline: 1.1083x; 1.1083x over previous
import functools

import jax
import jax.numpy as jnp
from jax import lax
from jax.experimental import pallas as pl
from jax.experimental.pallas import tpu as pltpu
from jax.experimental.pallas import tpu_sc as plsc

VOCAB = 100000
HIDDEN = 768
MAX_POS = 8192
BATCH = 4
SEQ = 8192

NC = 2
NS = 16
NW = NC * NS
L = 16

S_PER_W = SEQ // NW
CH = 16
HF = CH // 2
NCH = S_PER_W // CH
NH = NCH // 2
UNITS = HIDDEN // L

_mesh = plsc.VectorSubcoreMesh(
    core_axis_name="c", subcore_axis_name="s", num_cores=NC, num_subcores=NS
)


@functools.partial(
    pl.kernel,
    out_type=jax.ShapeDtypeStruct((BATCH, SEQ, HIDDEN), jnp.float32),
    mesh=_mesh,
    scratch_types=[
        pltpu.VMEM((BATCH, S_PER_W), jnp.int32),
        pltpu.VMEM((CH, HIDDEN), jnp.float32),
        pltpu.VMEM((CH, HIDDEN), jnp.float32),
    ] + [pltpu.VMEM((CH, HIDDEN), jnp.float32)] * 8
      + [pltpu.SemaphoreType.DMA] * 18,
)
def _embed(idx_hbm, tok_hbm, pos_hbm, out_hbm, idx_v, *bufs_and_sems):
    pbuf = bufs_and_sems[0:2]
    rbuf = bufs_and_sems[2:10]
    psem = bufs_and_sems[10:12]
    gsem = bufs_and_sems[12:20]
    ssem = bufs_and_sems[20:28]
    wid = lax.axis_index("s") * NC + lax.axis_index("c")
    s_base = wid * S_PER_W

    def gather(c, b, buf):
        return pltpu.async_copy(
            tok_hbm.at[idx_v.at[b, pl.ds(c * CH, CH)]], rbuf[buf], gsem[buf]
        )

    def gather_wait(buf):
        pltpu.make_async_copy(
            tok_hbm.at[idx_v.at[0, pl.ds(0, CH)]], rbuf[buf], gsem[buf]
        ).wait()

    def store_wait(buf):
        pltpu.make_async_copy(
            rbuf[buf], out_hbm.at[0, pl.ds(s_base, CH)], ssem[buf]
        ).wait()

    def pos_load(c, buf):
        return pltpu.async_copy(
            pos_hbm.at[pl.ds(s_base + c * CH, CH)], pbuf[buf], psem[buf]
        )

    pltpu.sync_copy(idx_hbm.at[0, pl.ds(s_base, S_PER_W)], idx_v.at[0])
    pos_load(0, 0)
    pos_load(1, 1)
    gather(0, 0, 0)
    pltpu.sync_copy(idx_hbm.at[1, pl.ds(s_base, S_PER_W)], idx_v.at[1])
    gather(0, 1, 1)
    pltpu.sync_copy(idx_hbm.at[2, pl.ds(s_base, S_PER_W)], idx_v.at[2])
    gather(0, 2, 2)
    pltpu.sync_copy(idx_hbm.at[3, pl.ds(s_base, S_PER_W)], idx_v.at[3])
    gather(0, 3, 3)

    def body(h, _):
        store_desc = [None] * 8
        gather_desc = [None] * 8
        for cs in range(2):
            g = cs
            og = 1 - cs
            c = 2 * h + cs
            s0 = s_base + c * CH

            if cs == 0:
                @pl.when(h > 0)
                def _():
                    for b in range(BATCH):
                        store_wait(og * 4 + b)
            else:
                for b in range(BATCH):
                    store_desc[og * 4 + b].wait()

            if cs == 0:
                for b in range(BATCH):
                    gather_desc[og * 4 + b] = gather(2 * h + 1, b, og * 4 + b)
            else:
                @pl.when(h < NH - 1)
                def _():
                    for b in range(BATCH):
                        gather(2 * h + 2, b, og * 4 + b)

            for b in range(BATCH):
                if cs == 0:
                    gather_wait(g * 4 + b)
                else:
                    gather_desc[g * 4 + b].wait()

            pltpu.make_async_copy(
                pos_hbm.at[pl.ds(s_base, CH)], pbuf[g], psem[g]
            ).wait()

            def add_rows(lo, hi, _g=g):
                @plsc.parallel_loop(lo, hi)
                def _(r):
                    for j in range(UNITS):
                        p = pbuf[_g][r, pl.ds(j * L, L)]
                        for b in range(BATCH):
                            plsc.addupdate(
                                rbuf[_g * 4 + b].at[r, pl.ds(j * L, L)], p
                            )

            for half in range(2):
                add_rows(half * HF, (half + 1) * HF)
                for b in range(BATCH):
                    pltpu.async_copy(
                        rbuf[g * 4 + b].at[pl.ds(half * HF, HF)],
                        out_hbm.at[b, pl.ds(s0 + half * HF, HF)],
                        ssem[g * 4 + b],
                    )
            for b in range(BATCH):
                store_desc[g * 4 + b] = pltpu.make_async_copy(
                    rbuf[g * 4 + b], out_hbm.at[b, pl.ds(s0, CH)],
                    ssem[g * 4 + b]
                )

            @pl.when(h < NH - 1)
            def _():
                pos_load(2 * h + 2 + cs, g)
        return 0

    lax.fori_loop(0, NH, body, 0)

    for b in range(BATCH):
        store_wait(4 + b)


def kernel(input_ids, tok_embed, pos_embed):
    return _embed(input_ids.astype(jnp.int32), tok_embed, pos_embed)
